# baseline (speedup 1.0000x reference)
import functools

import jax
import jax.numpy as jnp
from jax.experimental import pallas as pl
from jax.experimental.pallas import tpu as pltpu

_BLK = 256


def _router_body(x_ref, wt_ref, probs_ref, logits_ref, z_ref, *, inv_n):
    i = pl.program_id(0)
    xb = x_ref[...]
    logits = jax.lax.dot_general(
        xb, wt_ref[...], (((1,), (1,)), ((), ())),
        preferred_element_type=jnp.float32)

    absum = jnp.sum(jnp.abs(xb), axis=-1, keepdims=True)

    m = jnp.max(logits, axis=-1, keepdims=True)
    e = jnp.exp(logits - m)
    s = jnp.sum(e, axis=-1, keepdims=True)
    probs_ref[...] = e * (1.0 / s)

    keep = absum > 0
    logits_ref[...] = jnp.where(keep, logits, 0.0)

    n_e = jnp.float32(logits.shape[1])
    lse = jnp.where(keep, jnp.log(s) + m, jnp.log(n_e))
    part = jnp.sum(lse * lse) * inv_n

    @pl.when(i == 0)
    def _():
        z_ref[...] = jnp.zeros_like(z_ref)

    z_ref[...] = z_ref[...] + part


def kernel(x, W):
    b, s, d = x.shape
    n = b * s
    e = W.shape[1]
    xf = x.reshape(n, d)

    blk = _BLK
    body = functools.partial(_router_body, inv_n=1.0 / n)
    probs, logits, z = pl.pallas_call(
        body,
        grid=(n // blk,),
        in_specs=[
            pl.BlockSpec((blk, d), lambda i: (i, 0)),
            pl.BlockSpec((e, d), lambda i: (0, 0)),
        ],
        out_specs=[
            pl.BlockSpec((blk, e), lambda i: (i, 0)),
            pl.BlockSpec((blk, e), lambda i: (i, 0)),
            pl.BlockSpec((1, 1), lambda i: (0, 0)),
        ],
        out_shape=[
            jax.ShapeDtypeStruct((n, e), jnp.float32),
            jax.ShapeDtypeStruct((n, e), jnp.float32),
            jax.ShapeDtypeStruct((1, 1), jnp.float32),
        ],
    )(xf, W.T)
    return probs, logits, z[0, 0]


# speedup vs baseline: 1.4585x; 1.4585x over previous
import functools

import jax
import jax.numpy as jnp
from jax.experimental import pallas as pl
from jax.experimental.pallas import tpu as pltpu

_BLK = 1024


def _router_body(x_ref, wt_ref, probs_ref, logits_ref, z_ref, *, inv_n):
    i = pl.program_id(0)
    xb = x_ref[...]
    logits = jax.lax.dot_general(
        xb, wt_ref[...], (((1,), (1,)), ((), ())),
        preferred_element_type=jnp.float32)

    absum = jnp.sum(jnp.abs(xb), axis=-1, keepdims=True)

    m = jnp.max(logits, axis=-1, keepdims=True)
    e = jnp.exp(logits - m)
    s = jnp.sum(e, axis=-1, keepdims=True)
    probs_ref[...] = e * (1.0 / s)

    keep = absum > 0
    logits_ref[...] = jnp.where(keep, logits, 0.0)

    n_e = jnp.float32(logits.shape[1])
    lse = jnp.where(keep, jnp.log(s) + m, jnp.log(n_e))
    part = jnp.sum(lse * lse) * inv_n

    @pl.when(i == 0)
    def _():
        z_ref[...] = jnp.zeros_like(z_ref)

    z_ref[...] = z_ref[...] + part


def kernel(x, W):
    b, s, d = x.shape
    n = b * s
    e = W.shape[1]
    xf = x.reshape(n, d)

    blk = _BLK
    body = functools.partial(_router_body, inv_n=1.0 / n)
    probs, logits, z = pl.pallas_call(
        body,
        grid=(n // blk,),
        in_specs=[
            pl.BlockSpec((blk, d), lambda i: (i, 0)),
            pl.BlockSpec((e, d), lambda i: (0, 0)),
        ],
        out_specs=[
            pl.BlockSpec((blk, e), lambda i: (i, 0)),
            pl.BlockSpec((blk, e), lambda i: (i, 0)),
            pl.BlockSpec((1, 1), lambda i: (0, 0)),
        ],
        out_shape=[
            jax.ShapeDtypeStruct((n, e), jnp.float32),
            jax.ShapeDtypeStruct((n, e), jnp.float32),
            jax.ShapeDtypeStruct((1, 1), jnp.float32),
        ],
    )(xf, W.T)
    return probs, logits, z[0, 0]
